# baseline (speedup 1.0000x reference)
import functools

import jax
import jax.numpy as jnp
from jax.experimental import pallas as pl
from jax.experimental.pallas import tpu as pltpu

B = 1024
L = 64
MEM = 100000
K = 50
DELTA = 1e-3
BLK_M = 2048
M_PAD = ((MEM + BLK_M - 1) // BLK_M) * BLK_M


def _d2_body(emb_ref, qsq_ref, keys_ref, out_ref):
    i = pl.program_id(0)
    keys = keys_ref[...]
    mm = jax.lax.dot_general(
        emb_ref[...], keys, (((1,), (1,)), ((), ())),
        precision=jax.lax.Precision.DEFAULT,
        preferred_element_type=jnp.float32)
    k_sq = jnp.sum(keys * keys, axis=1)
    d2 = qsq_ref[...] + k_sq[None, :] - 2.0 * mm
    col = i * BLK_M + jax.lax.broadcasted_iota(jnp.int32, (1, BLK_M), 1)
    out_ref[...] = jnp.where(col >= MEM, jnp.float32(1e30), d2)


def _d2_matrix(emb, q_sq, keys_pad):
    grid = M_PAD // BLK_M
    return pl.pallas_call(
        _d2_body,
        grid=(grid,),
        in_specs=[
            pl.BlockSpec((B, L), lambda i: (0, 0)),
            pl.BlockSpec((B, 1), lambda i: (0, 0)),
            pl.BlockSpec((BLK_M, L), lambda i: (i, 0)),
        ],
        out_specs=pl.BlockSpec((B, BLK_M), lambda i: (0, i)),
        out_shape=jax.ShapeDtypeStruct((B, M_PAD), jnp.float32),
    )(emb, q_sq, keys_pad)


def kernel(state, W_repr1, b_repr1, W_repr2, b_repr2, W_fc1, b_fc1,
           W_policy, b_policy, W_vemb, b_vemb, memory_keys, memory_values):
    h = jax.nn.relu(state @ W_repr1.T + b_repr1)
    latent = h @ W_repr2.T + b_repr2
    out = jax.nn.relu(latent @ W_fc1.T + b_fc1)
    policy_logits = out @ W_policy.T + b_policy
    emb = out @ W_vemb.T + b_vemb

    q = jax.lax.stop_gradient(emb)
    q_sq = jnp.sum(q * q, axis=1, keepdims=True)
    keys_pad = jnp.pad(memory_keys, ((0, M_PAD - MEM), (0, 0)))
    d2 = _d2_matrix(q, q_sq, keys_pad)

    _, knn_idx = jax.lax.top_k(-d2, K)
    neighbors_repr = jnp.take(memory_keys, knn_idx, axis=0)
    neighbors_value = jnp.take(memory_values, knn_idx, axis=0)
    diff = emb[:, None, :] - neighbors_repr
    dists = jnp.sqrt(jnp.maximum(jnp.sum(diff * diff, axis=-1), 1e-12))
    kern = 1.0 / (dists + DELTA)
    w = kern / jnp.sum(kern)
    value_logits = jnp.sum(neighbors_value * w, axis=1)
    return policy_logits, value_logits


# speedup vs baseline: 8.2196x; 8.2196x over previous
import functools

import jax
import jax.numpy as jnp
from jax import lax
from jax.experimental import pallas as pl
from jax.experimental.pallas import tpu as pltpu
from jax.experimental.pallas import tpu_sc as plsc

B = 1024
L = 64
MEM = 100000
K = 50
DELTA = 1e-3

BLK_M = 2048
M_PAD = 100352
G = 128
P = M_PAD // G
SUP = 8
NSUP = P // SUP
NSUP_PAD = 112
NW = 16
ROWS_W = B // NW

CAP_A = 256
CAP_C = 512
SEG_C = 128
NCHUNK = 64
SEL_CAP = 80
P_ROW = 896
BIG = 1e30


def _lane():
    return lax.broadcasted_iota(jnp.int32, (16,), 0)


def _stage1_body(state_ref, w1_ref, b1_ref, w2_ref, b2_ref, wf_ref, bf_ref,
                 wp_ref, bp_ref, wv_ref, bv_ref, keys_ref,
                 d2_ref, mins_ref, pol_ref, emb_scr, qsq_scr,
                 macc_ref, sacc_ref):
    i = pl.program_id(0)

    @pl.when(i == 0)
    def _():
        dn = (((1,), (1,)), ((), ()))
        hp = jax.lax.Precision.HIGHEST
        h = jnp.maximum(
            lax.dot_general(state_ref[...], w1_ref[...], dn,
                            preferred_element_type=jnp.float32) + b1_ref[...], 0.0)
        latent = lax.dot_general(h, w2_ref[...], dn,
                                 preferred_element_type=jnp.float32) + b2_ref[...]
        o = jnp.maximum(
            lax.dot_general(latent, wf_ref[...], dn,
                            preferred_element_type=jnp.float32) + bf_ref[...], 0.0)
        pol_ref[...] = lax.dot_general(o, wp_ref[...], dn,
                                       preferred_element_type=jnp.float32) + bp_ref[...]
        emb = lax.dot_general(o, wv_ref[...], dn,
                              preferred_element_type=jnp.float32) + bv_ref[...]
        emb_scr[...] = emb
        qsq_scr[...] = jnp.sum(emb * emb, axis=1, keepdims=True)

    keys = keys_ref[...]
    mm = lax.dot_general(emb_scr[...], keys, (((1,), (1,)), ((), ())),
                         preferred_element_type=jnp.float32)
    k_sq = jnp.sum(keys * keys, axis=1)
    d2 = qsq_scr[...] + k_sq[None, :] - 2.0 * mm
    col = i * BLK_M + lax.broadcasted_iota(jnp.int32, (1, BLK_M), 1)
    d2 = jnp.where(col >= MEM, jnp.float32(BIG), d2)
    d2_ref[...] = d2

    cols = [jnp.min(d2[:, j * G:(j + 1) * G], axis=1, keepdims=True)
            for j in range(BLK_M // G)]
    mins = jnp.concatenate(cols, axis=1)
    m2 = jnp.concatenate(
        [jnp.min(mins[:, :SUP], axis=1, keepdims=True),
         jnp.min(mins[:, SUP:], axis=1, keepdims=True)], axis=1)
    macc_ref[...] = jnp.concatenate([macc_ref[:, 16:], mins], axis=1)
    sacc_ref[...] = jnp.concatenate([sacc_ref[:, 2:], m2], axis=1)

    @pl.when(jnp.logical_and(i % 8 == 7, i < 48))
    def _():
        mins_ref[:, pl.ds((i // 8) * 128, 128)] = macc_ref[...]

    @pl.when(i == 48)
    def _():
        mins_ref[:, pl.ds(768, 128)] = jnp.concatenate(
            [macc_ref[:, 112:], sacc_ref[:, 30:],
             jnp.full((B, 14), BIG, jnp.float32)], axis=1)


def _stage1(state, W1, b1, W2, b2, Wf, bf, Wp, bp, Wv, bv, keys_pad):
    grid = M_PAD // BLK_M
    return pl.pallas_call(
        _stage1_body,
        grid=(grid,),
        in_specs=[
            pl.BlockSpec((B, 4), lambda i: (0, 0)),
            pl.BlockSpec((L, 4), lambda i: (0, 0)),
            pl.BlockSpec((1, L), lambda i: (0, 0)),
            pl.BlockSpec((L, L), lambda i: (0, 0)),
            pl.BlockSpec((1, L), lambda i: (0, 0)),
            pl.BlockSpec((L, L), lambda i: (0, 0)),
            pl.BlockSpec((1, L), lambda i: (0, 0)),
            pl.BlockSpec((2, L), lambda i: (0, 0)),
            pl.BlockSpec((1, 2), lambda i: (0, 0)),
            pl.BlockSpec((L, L), lambda i: (0, 0)),
            pl.BlockSpec((1, L), lambda i: (0, 0)),
            pl.BlockSpec((BLK_M, L), lambda i: (i, 0)),
        ],
        out_specs=[
            pl.BlockSpec((B, BLK_M), lambda i: (0, i)),
            pl.BlockSpec((B, P_ROW), lambda i: (0, 0)),
            pl.BlockSpec((B, 2), lambda i: (0, 0)),
        ],
        out_shape=[
            jax.ShapeDtypeStruct((B, M_PAD), jnp.float32),
            jax.ShapeDtypeStruct((B, P_ROW), jnp.float32),
            jax.ShapeDtypeStruct((B, 2), jnp.float32),
        ],
        scratch_shapes=[
            pltpu.VMEM((B, L), jnp.float32),
            pltpu.VMEM((B, 1), jnp.float32),
            pltpu.VMEM((B, 128), jnp.float32),
            pltpu.VMEM((B, 128), jnp.float32),
        ],
    )(state, W1, b1.reshape(1, L), W2, b2.reshape(1, L), Wf, bf.reshape(1, L),
      Wp, bp.reshape(1, 2), Wv, bv.reshape(1, L), keys_pad)


def _unmap(u):
    top = jnp.uint32(0x80000000)
    return jnp.where(
        u >= top,
        lax.bitcast_convert_type(u - top, jnp.float32),
        lax.bitcast_convert_type(~u, jnp.float32))


def _map_f32(x):
    u = lax.bitcast_convert_type(x, jnp.uint32)
    return jnp.where(u < jnp.uint32(0x80000000), u + jnp.uint32(0x80000000), ~u)


def _popc(m):
    return plsc.all_reduce_population_count(m)[0]


def _count_le(buf_ref, base0, ntrip, t):
    def body(j, acc):
        v = buf_ref[pl.ds(base0 + j * 16, 16)]
        return acc + jnp.where(v <= t, 1.0, 0.0).astype(jnp.float32)
    acc = lax.fori_loop(0, ntrip, body, jnp.zeros((16,), jnp.float32))
    return jnp.sum(acc).astype(jnp.int32)


def _count_lt(buf_ref, base0, ntrip, t):
    def body(j, acc):
        v = buf_ref[pl.ds(base0 + j * 16, 16)]
        return acc + jnp.where(v < t, 1.0, 0.0).astype(jnp.float32)
    acc = lax.fori_loop(0, ntrip, body, jnp.zeros((16,), jnp.float32))
    return jnp.sum(acc).astype(jnp.int32)


def _bisect_kth(buf_ref, base0, ntrip, k):
    def body(_, lohi):
        lo, hi = lohi
        mid = lo + (hi - lo) // jnp.uint32(2)
        c = _count_le(buf_ref, base0, ntrip, _unmap(mid))
        return jnp.where(c >= k, lo, mid), jnp.where(c >= k, mid, hi)

    lo0 = jnp.uint32(0)
    hi0 = _map_f32(jnp.float32(BIG))
    _, hi = lax.fori_loop(0, 32, body, (lo0, hi0))
    return _unmap(hi)


def _sc_body(mins_hbm, d2v_hbm, mv_hbm, out_hbm,
             minsrow, chunks, bufAv, bufAg, bufCv, bufCi,
             selg, ggidx, selv, seli, selvals, outbuf, sem, sem2):
    wid = lax.axis_index("s")

    def row_body(i, carry):
        r = wid * ROWS_W + i
        pltpu.sync_copy(mins_hbm.at[r], minsrow)

        T1 = _bisect_kth(minsrow, P, NSUP_PAD // 16, K)

        for q in range(CAP_A // 16):
            bufAv[pl.ds(q * 16, 16)] = jnp.full((16,), BIG, jnp.float32)

        def a2(kk, cnt):
            v = minsrow[pl.ds(kk * 16, 16)]
            g = kk * 16 + _lane()
            m = v <= T1
            base = jnp.minimum(cnt, CAP_A - 16)
            plsc.store_compressed(bufAv.at[pl.ds(base, 16)], v, mask=m)
            plsc.store_compressed(bufAg.at[pl.ds(base, 16)], g, mask=m)
            return cnt + _popc(m)
        cntA = lax.fori_loop(0, P // 16, a2, jnp.int32(0))

        ntripA = (cntA + 15) // 16
        T50m = _bisect_kth(bufAv, 0, ntripA, K)

        for q in range(SEL_CAP // 16):
            selg[pl.ds(q * 16, 16)] = jnp.full((16,), P - 1, jnp.int32)

        def a4(kk, cnt):
            v = bufAv[pl.ds(kk * 16, 16)]
            g = bufAg[pl.ds(kk * 16, 16)]
            m = v <= T50m
            base = jnp.minimum(cnt, SEL_CAP - 16)
            plsc.store_compressed(selg.at[pl.ds(base, 16)], g, mask=m)
            return cnt + _popc(m)
        lax.fori_loop(0, ntripA, a4, jnp.int32(0))

        rbase = r * P
        for q in range(NCHUNK // 16):
            ggidx[pl.ds(q * 16, 16)] = rbase + selg[pl.ds(q * 16, 16)]
        pltpu.async_copy(d2v_hbm.at[ggidx], chunks, sem).wait()

        for q in range(CAP_C // 16):
            bufCv[pl.ds(q * 16, 16)] = jnp.full((16,), BIG, jnp.float32)

        cnts = [jnp.int32(0)] * 4
        t = 0
        for q in range(NCHUNK // 16):
            gvec = selg[pl.ds(q * 16, 16)]
            for l in range(16):
                ebase = gvec[l] * G
                for j in range(G // 16):
                    v = chunks[q * 16 + l, pl.ds(j * 16, 16)]
                    ei = ebase + j * 16 + _lane()
                    m = v <= T50m
                    ch = t % 4
                    t += 1
                    base = SEG_C * ch + jnp.minimum(cnts[ch], SEG_C - 16)
                    plsc.store_compressed(bufCv.at[pl.ds(base, 16)], v, mask=m)
                    plsc.store_compressed(bufCi.at[pl.ds(base, 16)], ei, mask=m)
                    cnts[ch] = cnts[ch] + _popc(m)

        vb = (cnts[0] + 15) // 16
        for ch in range(1, 4):
            nk = (cnts[ch] + 15) // 16

            def mv(j, vb0, _ch=ch):
                bufCv[pl.ds(16 * (vb0 + j), 16)] = bufCv[pl.ds(SEG_C * _ch + 16 * j, 16)]
                bufCi[pl.ds(16 * (vb0 + j), 16)] = bufCi[pl.ds(SEG_C * _ch + 16 * j, 16)]
                return vb0
            lax.fori_loop(0, nk, mv, vb)
            vb = vb + nk
        cntC = cnts[0] + cnts[1] + cnts[2] + cnts[3]

        ntripC = vb
        tstar = _bisect_kth(bufCv, 0, ntripC, K)
        c_le = _count_le(bufCv, 0, ntripC, tstar)

        def tie_cut(_):
            c_lt = _count_lt(bufCv, 0, ntripC, tstar)
            kk = K - c_lt

            def cbody(_, lohi):
                lo, hi = lohi
                mid = lo + (hi - lo) // 2

                def cb(j, c):
                    v = bufCv[pl.ds(j * 16, 16)]
                    ei = bufCi[pl.ds(j * 16, 16)]
                    return c + _popc((v == tstar) & (ei <= mid))
                c = lax.fori_loop(0, ntripC, cb, jnp.int32(0))
                return jnp.where(c >= kk, lo, mid), jnp.where(c >= kk, mid, hi)

            _, hi = lax.fori_loop(0, 18, cbody, (jnp.int32(-1), jnp.int32(M_PAD)))
            return hi
        cut = lax.cond(c_le == K, lambda _: jnp.int32(M_PAD), tie_cut, 0)

        for q in range(SEL_CAP // 16):
            selv[pl.ds(q * 16, 16)] = jnp.full((16,), BIG, jnp.float32)
            seli[pl.ds(q * 16, 16)] = jnp.zeros((16,), jnp.int32)

        def c4(j, cnt):
            v = bufCv[pl.ds(j * 16, 16)]
            ei = bufCi[pl.ds(j * 16, 16)]
            m = (v < tstar) | ((v == tstar) & (ei <= cut))
            base = jnp.minimum(cnt, SEL_CAP - 16)
            plsc.store_compressed(selv.at[pl.ds(base, 16)], v, mask=m)
            plsc.store_compressed(seli.at[pl.ds(base, 16)], ei, mask=m)
            return cnt + _popc(m)
        lax.fori_loop(0, ntripC, c4, jnp.int32(0))

        pltpu.async_copy(mv_hbm.at[seli], selvals, sem2).wait()
        accw = jnp.zeros((16,), jnp.float32)
        acck = jnp.zeros((16,), jnp.float32)
        for q in range(SEL_CAP // 16):
            d2c = jnp.maximum(selv[pl.ds(q * 16, 16)], jnp.float32(1e-12))
            ub = lax.bitcast_convert_type(d2c, jnp.int32)
            s = lax.bitcast_convert_type(0x1FBD1DF5 + (ub >> 1), jnp.float32)
            s = 0.5 * (s + d2c / s)
            s = 0.5 * (s + d2c / s)
            s = 0.5 * (s + d2c / s)
            kern = 1.0 / (s + DELTA)
            accw = accw + kern * selvals[pl.ds(q * 16, 16)]
            acck = acck + kern
        wsum = jnp.sum(accw)
        ksum = jnp.sum(acck)
        ln = _lane()
        res = jnp.where(ln == 0, wsum, jnp.where(ln == 1, ksum, 0.0))
        res = jnp.where(ln == 2, cntA.astype(jnp.float32), res)
        res = jnp.where(ln == 3, cntC.astype(jnp.float32), res)
        res = jnp.where(ln == 4, T1, res)
        res = jnp.where(ln == 5, T50m, res)
        res = jnp.where(ln == 6, tstar, res)
        res = jnp.where(ln == 7, c_le.astype(jnp.float32), res)
        res = jnp.where(ln == 8, cut.astype(jnp.float32), res)
        outbuf[pl.ds(i * 16, 16)] = res
        return carry

    lax.fori_loop(0, ROWS_W, row_body, jnp.int32(0))
    pltpu.sync_copy(outbuf, out_hbm.at[pl.ds(wid * ROWS_W * 16, ROWS_W * 16)])


@functools.partial(
    pl.kernel,
    out_type=jax.ShapeDtypeStruct((B * 16,), jnp.float32),
    mesh=plsc.VectorSubcoreMesh(core_axis_name="c", subcore_axis_name="s", num_cores=1),
    compiler_params=pltpu.CompilerParams(needs_layout_passes=False),
    scratch_types=[
        pltpu.VMEM((P_ROW,), jnp.float32),
        pltpu.VMEM((NCHUNK, G), jnp.float32),
        pltpu.VMEM((CAP_A,), jnp.float32),
        pltpu.VMEM((CAP_A,), jnp.int32),
        pltpu.VMEM((CAP_C,), jnp.float32),
        pltpu.VMEM((CAP_C,), jnp.int32),
        pltpu.VMEM((SEL_CAP,), jnp.int32),
        pltpu.VMEM((NCHUNK,), jnp.int32),
        pltpu.VMEM((SEL_CAP,), jnp.float32),
        pltpu.VMEM((SEL_CAP,), jnp.int32),
        pltpu.VMEM((SEL_CAP,), jnp.float32),
        pltpu.VMEM((ROWS_W * 16,), jnp.float32),
        pltpu.SemaphoreType.DMA,
        pltpu.SemaphoreType.DMA,
    ],
)
def _sc_topk(mins_hbm, d2v_hbm, mv_hbm, out_hbm, *scratch):
    _sc_body(mins_hbm, d2v_hbm, mv_hbm, out_hbm, *scratch)


def kernel(state, W_repr1, b_repr1, W_repr2, b_repr2, W_fc1, b_fc1,
           W_policy, b_policy, W_vemb, b_vemb, memory_keys, memory_values):
    keys_pad = jnp.pad(memory_keys, ((0, M_PAD - MEM), (0, 0)))
    d2, mins, policy_logits = _stage1(
        state, W_repr1, b_repr1, W_repr2, b_repr2, W_fc1, b_fc1,
        W_policy, b_policy, W_vemb, b_vemb, keys_pad)
    d2v = d2.reshape(B * P, G)
    out = _sc_topk(mins, d2v, memory_values).reshape(B, 16)
    value_logits = out[:, 0] / jnp.sum(out[:, 1])
    return policy_logits, value_logits


# speedup vs baseline: 12.5043x; 1.5213x over previous
import functools

import jax
import jax.numpy as jnp
from jax import lax
from jax.experimental import pallas as pl
from jax.experimental.pallas import tpu as pltpu
from jax.experimental.pallas import tpu_sc as plsc

B = 1024
L = 64
MEM = 100000
K = 50
DELTA = 1e-3

BLK_M = 2048
M_PAD = 100352
G = 128
P = M_PAD // G
SUP = 8
NSUP = P // SUP
NSUP_PAD = 112
NW = 32
ROWS_W = B // NW

CAP_A = 256
CAP_C = 512
SEG_C = 128
NCHUNK = 64
SEL_CAP = 80
P_ROW = 896
BIG = 1e30


def _lane():
    return lax.broadcasted_iota(jnp.int32, (16,), 0)


def _stage1_body(state_ref, w1_ref, b1_ref, w2_ref, b2_ref, wf_ref, bf_ref,
                 wp_ref, bp_ref, wv_ref, bv_ref, keys_ref,
                 d2_ref, mins_ref, pol_ref, emb_scr, qsq_scr,
                 macc_ref, sacc_ref):
    i = pl.program_id(0)

    @pl.when(i == 0)
    def _():
        dn = (((1,), (1,)), ((), ()))
        hp = jax.lax.Precision.HIGHEST
        h = jnp.maximum(
            lax.dot_general(state_ref[...], w1_ref[...], dn,
                            preferred_element_type=jnp.float32) + b1_ref[...], 0.0)
        latent = lax.dot_general(h, w2_ref[...], dn,
                                 preferred_element_type=jnp.float32) + b2_ref[...]
        o = jnp.maximum(
            lax.dot_general(latent, wf_ref[...], dn,
                            preferred_element_type=jnp.float32) + bf_ref[...], 0.0)
        pol_ref[...] = lax.dot_general(o, wp_ref[...], dn,
                                       preferred_element_type=jnp.float32) + bp_ref[...]
        emb = lax.dot_general(o, wv_ref[...], dn,
                              preferred_element_type=jnp.float32) + bv_ref[...]
        emb_scr[...] = emb
        qsq_scr[...] = jnp.sum(emb * emb, axis=1, keepdims=True)

    keys = keys_ref[...]
    mm = lax.dot_general(emb_scr[...], keys, (((1,), (1,)), ((), ())),
                         preferred_element_type=jnp.float32)
    k_sq = jnp.sum(keys * keys, axis=1)
    d2 = qsq_scr[...] + k_sq[None, :] - 2.0 * mm
    col = i * BLK_M + lax.broadcasted_iota(jnp.int32, (1, BLK_M), 1)
    d2 = jnp.where(col >= MEM, jnp.float32(BIG), d2)
    d2_ref[...] = d2

    cols = [jnp.min(d2[:, j * G:(j + 1) * G], axis=1, keepdims=True)
            for j in range(BLK_M // G)]
    mins = jnp.concatenate(cols, axis=1)
    m2 = jnp.concatenate(
        [jnp.min(mins[:, :SUP], axis=1, keepdims=True),
         jnp.min(mins[:, SUP:], axis=1, keepdims=True)], axis=1)
    macc_ref[...] = jnp.concatenate([macc_ref[:, 16:], mins], axis=1)
    sacc_ref[...] = jnp.concatenate([sacc_ref[:, 2:], m2], axis=1)

    @pl.when(jnp.logical_and(i % 8 == 7, i < 48))
    def _():
        mins_ref[:, pl.ds((i // 8) * 128, 128)] = macc_ref[...]

    @pl.when(i == 48)
    def _():
        mins_ref[:, pl.ds(768, 128)] = jnp.concatenate(
            [macc_ref[:, 112:], sacc_ref[:, 30:],
             jnp.full((B, 14), BIG, jnp.float32)], axis=1)


def _stage1(state, W1, b1, W2, b2, Wf, bf, Wp, bp, Wv, bv, keys_pad):
    grid = M_PAD // BLK_M
    return pl.pallas_call(
        _stage1_body,
        grid=(grid,),
        in_specs=[
            pl.BlockSpec((B, 4), lambda i: (0, 0)),
            pl.BlockSpec((L, 4), lambda i: (0, 0)),
            pl.BlockSpec((1, L), lambda i: (0, 0)),
            pl.BlockSpec((L, L), lambda i: (0, 0)),
            pl.BlockSpec((1, L), lambda i: (0, 0)),
            pl.BlockSpec((L, L), lambda i: (0, 0)),
            pl.BlockSpec((1, L), lambda i: (0, 0)),
            pl.BlockSpec((2, L), lambda i: (0, 0)),
            pl.BlockSpec((1, 2), lambda i: (0, 0)),
            pl.BlockSpec((L, L), lambda i: (0, 0)),
            pl.BlockSpec((1, L), lambda i: (0, 0)),
            pl.BlockSpec((BLK_M, L), lambda i: (i, 0)),
        ],
        out_specs=[
            pl.BlockSpec((B, BLK_M), lambda i: (0, i)),
            pl.BlockSpec((B, P_ROW), lambda i: (0, 0)),
            pl.BlockSpec((B, 2), lambda i: (0, 0)),
        ],
        out_shape=[
            jax.ShapeDtypeStruct((B, M_PAD), jnp.float32),
            jax.ShapeDtypeStruct((B, P_ROW), jnp.float32),
            jax.ShapeDtypeStruct((B, 2), jnp.float32),
        ],
        scratch_shapes=[
            pltpu.VMEM((B, L), jnp.float32),
            pltpu.VMEM((B, 1), jnp.float32),
            pltpu.VMEM((B, 128), jnp.float32),
            pltpu.VMEM((B, 128), jnp.float32),
        ],
    )(state, W1, b1.reshape(1, L), W2, b2.reshape(1, L), Wf, bf.reshape(1, L),
      Wp, bp.reshape(1, 2), Wv, bv.reshape(1, L), keys_pad)


def _unmap(u):
    top = jnp.uint32(0x80000000)
    return jnp.where(
        u >= top,
        lax.bitcast_convert_type(u - top, jnp.float32),
        lax.bitcast_convert_type(~u, jnp.float32))


def _map_f32(x):
    u = lax.bitcast_convert_type(x, jnp.uint32)
    return jnp.where(u < jnp.uint32(0x80000000), u + jnp.uint32(0x80000000), ~u)


def _popc(m):
    return plsc.all_reduce_population_count(m)[0]


def _count_le(buf_ref, base0, ntrip, t):
    def body(j, acc):
        v = buf_ref[pl.ds(base0 + j * 16, 16)]
        return acc + jnp.where(v <= t, 1.0, 0.0).astype(jnp.float32)
    acc = jnp.zeros((16,), jnp.float32)
    if isinstance(ntrip, int):
        for j in range(ntrip):
            acc = body(j, acc)
    else:
        acc = lax.fori_loop(0, ntrip, body, acc)
    return jnp.sum(acc).astype(jnp.int32)


def _count_lt(buf_ref, base0, ntrip, t):
    def body(j, acc):
        v = buf_ref[pl.ds(base0 + j * 16, 16)]
        return acc + jnp.where(v < t, 1.0, 0.0).astype(jnp.float32)
    acc = lax.fori_loop(0, ntrip, body, jnp.zeros((16,), jnp.float32))
    return jnp.sum(acc).astype(jnp.int32)


def _bisect_kth(buf_ref, base0, ntrip, k):
    def body(_, lohi):
        lo, hi = lohi
        mid = lo + (hi - lo) // jnp.uint32(2)
        c = _count_le(buf_ref, base0, ntrip, _unmap(mid))
        return jnp.where(c >= k, lo, mid), jnp.where(c >= k, mid, hi)

    lo0 = jnp.uint32(0)
    hi0 = _map_f32(jnp.float32(BIG))
    _, hi = lax.fori_loop(0, 32, body, (lo0, hi0))
    return _unmap(hi)


def _sc_body(mins_hbm, d2v_hbm, mv_hbm, out_hbm,
             minsall, chunks, bufAv, bufAg, bufCv, bufCi,
             selg, ggidx, selv, seli, selvals, outbuf, sem, sem2):
    wid = lax.axis_index("s") * 2 + lax.axis_index("c")
    pltpu.sync_copy(
        mins_hbm.at[pl.ds(wid * ROWS_W * P_ROW, ROWS_W * P_ROW)], minsall)

    def row_body(i, carry):
        r = wid * ROWS_W + i
        rb = i * P_ROW

        T1 = _bisect_kth(minsall, rb + P, NSUP_PAD // 16, K)

        for q in range(CAP_A // 16):
            bufAv[pl.ds(q * 16, 16)] = jnp.full((16,), BIG, jnp.float32)

        def a2(kk, cnt):
            v = minsall[pl.ds(rb + kk * 16, 16)]
            g = kk * 16 + _lane()
            m = v <= T1
            base = jnp.minimum(cnt, CAP_A - 16)
            plsc.store_compressed(bufAv.at[pl.ds(base, 16)], v, mask=m)
            plsc.store_compressed(bufAg.at[pl.ds(base, 16)], g, mask=m)
            return cnt + _popc(m)
        cntA = lax.fori_loop(0, P // 16, a2, jnp.int32(0))

        ntripA = (cntA + 15) // 16
        T50m = _bisect_kth(bufAv, 0, CAP_A // 16, K)

        for q in range(SEL_CAP // 16):
            selg[pl.ds(q * 16, 16)] = jnp.full((16,), P - 1, jnp.int32)

        def a4(kk, cnt):
            v = bufAv[pl.ds(kk * 16, 16)]
            g = bufAg[pl.ds(kk * 16, 16)]
            m = v <= T50m
            base = jnp.minimum(cnt, SEL_CAP - 16)
            plsc.store_compressed(selg.at[pl.ds(base, 16)], g, mask=m)
            return cnt + _popc(m)
        lax.fori_loop(0, ntripA, a4, jnp.int32(0))

        rbase = r * P
        for q in range(NCHUNK // 16):
            ggidx[pl.ds(q * 16, 16)] = rbase + selg[pl.ds(q * 16, 16)]
        pltpu.async_copy(d2v_hbm.at[ggidx], chunks, sem).wait()

        for q in range(CAP_C // 16):
            bufCv[pl.ds(q * 16, 16)] = jnp.full((16,), BIG, jnp.float32)

        cnts = [jnp.int32(0)] * 4
        t = 0
        for q in range(NCHUNK // 16):
            gvec = selg[pl.ds(q * 16, 16)]
            for l in range(16):
                ebase = gvec[l] * G
                for j in range(G // 16):
                    v = chunks[q * 16 + l, pl.ds(j * 16, 16)]
                    ei = ebase + j * 16 + _lane()
                    m = v <= T50m
                    ch = t % 4
                    t += 1
                    base = SEG_C * ch + jnp.minimum(cnts[ch], SEG_C - 16)
                    plsc.store_compressed(bufCv.at[pl.ds(base, 16)], v, mask=m)
                    plsc.store_compressed(bufCi.at[pl.ds(base, 16)], ei, mask=m)
                    cnts[ch] = cnts[ch] + _popc(m)

        vb = (cnts[0] + 15) // 16
        for ch in range(1, 4):
            nk = (cnts[ch] + 15) // 16

            def mv(j, vb0, _ch=ch):
                bufCv[pl.ds(16 * (vb0 + j), 16)] = bufCv[pl.ds(SEG_C * _ch + 16 * j, 16)]
                bufCi[pl.ds(16 * (vb0 + j), 16)] = bufCi[pl.ds(SEG_C * _ch + 16 * j, 16)]
                return vb0
            lax.fori_loop(0, nk, mv, vb)
            vb = vb + nk
        cntC = cnts[0] + cnts[1] + cnts[2] + cnts[3]

        ntripC = vb
        tstar = _bisect_kth(bufCv, 0, ntripC, K)
        c_le = _count_le(bufCv, 0, ntripC, tstar)

        def tie_cut(_):
            c_lt = _count_lt(bufCv, 0, ntripC, tstar)
            kk = K - c_lt

            def cbody(_, lohi):
                lo, hi = lohi
                mid = lo + (hi - lo) // 2

                def cb(j, c):
                    v = bufCv[pl.ds(j * 16, 16)]
                    ei = bufCi[pl.ds(j * 16, 16)]
                    return c + _popc((v == tstar) & (ei <= mid))
                c = lax.fori_loop(0, ntripC, cb, jnp.int32(0))
                return jnp.where(c >= kk, lo, mid), jnp.where(c >= kk, mid, hi)

            _, hi = lax.fori_loop(0, 18, cbody, (jnp.int32(-1), jnp.int32(M_PAD)))
            return hi
        cut = lax.cond(c_le == K, lambda _: jnp.int32(M_PAD), tie_cut, 0)

        for q in range(SEL_CAP // 16):
            selv[pl.ds(q * 16, 16)] = jnp.full((16,), BIG, jnp.float32)
            seli[pl.ds(q * 16, 16)] = jnp.zeros((16,), jnp.int32)

        def c4(j, cnt):
            v = bufCv[pl.ds(j * 16, 16)]
            ei = bufCi[pl.ds(j * 16, 16)]
            m = (v < tstar) | ((v == tstar) & (ei <= cut))
            base = jnp.minimum(cnt, SEL_CAP - 16)
            plsc.store_compressed(selv.at[pl.ds(base, 16)], v, mask=m)
            plsc.store_compressed(seli.at[pl.ds(base, 16)], ei, mask=m)
            return cnt + _popc(m)
        lax.fori_loop(0, ntripC, c4, jnp.int32(0))

        pltpu.async_copy(mv_hbm.at[seli], selvals, sem2).wait()
        accw = jnp.zeros((16,), jnp.float32)
        acck = jnp.zeros((16,), jnp.float32)
        for q in range(SEL_CAP // 16):
            d2c = jnp.maximum(selv[pl.ds(q * 16, 16)], jnp.float32(1e-12))
            ub = lax.bitcast_convert_type(d2c, jnp.int32)
            s = lax.bitcast_convert_type(0x1FBD1DF5 + (ub >> 1), jnp.float32)
            s = 0.5 * (s + d2c / s)
            s = 0.5 * (s + d2c / s)
            s = 0.5 * (s + d2c / s)
            kern = 1.0 / (s + DELTA)
            accw = accw + kern * selvals[pl.ds(q * 16, 16)]
            acck = acck + kern
        wsum = jnp.sum(accw)
        ksum = jnp.sum(acck)
        ln = _lane()
        res = jnp.where(ln == 0, wsum, jnp.where(ln == 1, ksum, 0.0))
        res = jnp.where(ln == 2, cntA.astype(jnp.float32), res)
        res = jnp.where(ln == 3, cntC.astype(jnp.float32), res)
        res = jnp.where(ln == 4, T1, res)
        res = jnp.where(ln == 5, T50m, res)
        res = jnp.where(ln == 6, tstar, res)
        res = jnp.where(ln == 7, c_le.astype(jnp.float32), res)
        res = jnp.where(ln == 8, cut.astype(jnp.float32), res)
        outbuf[pl.ds(i * 16, 16)] = res
        return carry

    lax.fori_loop(0, ROWS_W, row_body, jnp.int32(0))
    pltpu.sync_copy(outbuf, out_hbm.at[pl.ds(wid * ROWS_W * 16, ROWS_W * 16)])


@functools.partial(
    pl.kernel,
    out_type=jax.ShapeDtypeStruct((B * 16,), jnp.float32),
    mesh=plsc.VectorSubcoreMesh(core_axis_name="c", subcore_axis_name="s"),
    compiler_params=pltpu.CompilerParams(needs_layout_passes=False),
    scratch_types=[
        pltpu.VMEM((ROWS_W * P_ROW,), jnp.float32),
        pltpu.VMEM((NCHUNK, G), jnp.float32),
        pltpu.VMEM((CAP_A,), jnp.float32),
        pltpu.VMEM((CAP_A,), jnp.int32),
        pltpu.VMEM((CAP_C,), jnp.float32),
        pltpu.VMEM((CAP_C,), jnp.int32),
        pltpu.VMEM((SEL_CAP,), jnp.int32),
        pltpu.VMEM((NCHUNK,), jnp.int32),
        pltpu.VMEM((SEL_CAP,), jnp.float32),
        pltpu.VMEM((SEL_CAP,), jnp.int32),
        pltpu.VMEM((SEL_CAP,), jnp.float32),
        pltpu.VMEM((ROWS_W * 16,), jnp.float32),
        pltpu.SemaphoreType.DMA,
        pltpu.SemaphoreType.DMA,
    ],
)
def _sc_topk(mins_hbm, d2v_hbm, mv_hbm, out_hbm, *scratch):
    _sc_body(mins_hbm, d2v_hbm, mv_hbm, out_hbm, *scratch)


def kernel(state, W_repr1, b_repr1, W_repr2, b_repr2, W_fc1, b_fc1,
           W_policy, b_policy, W_vemb, b_vemb, memory_keys, memory_values):
    keys_pad = jnp.pad(memory_keys, ((0, M_PAD - MEM), (0, 0)))
    d2, mins, policy_logits = _stage1(
        state, W_repr1, b_repr1, W_repr2, b_repr2, W_fc1, b_fc1,
        W_policy, b_policy, W_vemb, b_vemb, keys_pad)
    d2v = d2.reshape(B * P, G)
    out = _sc_topk(mins.reshape(B * P_ROW), d2v, memory_values).reshape(B, 16)
    value_logits = out[:, 0] / jnp.sum(out[:, 1])
    return policy_logits, value_logits
